# Initial kernel scaffold; baseline (speedup 1.0000x reference)
import jax
import jax.numpy as jnp
from jax.experimental import pallas as pl


def kernel(tl_heat, br_heat, tl_embd, br_embd, tl_offs, br_offs):
    raise NotImplementedError("write your pallas kernel here")


# speedup vs baseline: 5.5475x; 5.5475x over previous
import jax
import jax.numpy as jnp
from jax.experimental import pallas as pl
from jax.experimental.pallas import tpu as pltpu

_K = 100
_NUM_DETS = 1000
_AE_THRESH = 0.5
_C = 80
_H = 128
_W = 128
_BIG = 1 << 30


def _phase1(heat_ref, embd_ref, offx_ref, offy_ref, attr_col_ref,
            nms_ref, rowmax_ref):
  x = heat_ref[...]
  s = 1.0 / (1.0 + jnp.exp(-x))
  zrow = jnp.zeros((_C, 1, _W), jnp.float32)
  up = jnp.concatenate([s[:, 1:, :], zrow], axis=1)
  dn = jnp.concatenate([zrow, s[:, :-1, :]], axis=1)
  v = jnp.maximum(jnp.maximum(s, up), dn)
  zcol = jnp.zeros((_C, _H, 1), jnp.float32)
  lt = jnp.concatenate([v[:, :, 1:], zcol], axis=2)
  rt = jnp.concatenate([zcol, v[:, :, :-1]], axis=2)
  y = jnp.maximum(jnp.maximum(v, lt), rt)
  nms = jnp.where(s == y, s, 0.0)
  nms_ref[...] = nms.reshape(_C * _H, _W)
  rowmax_ref[...] = jnp.max(nms, axis=2)

  rid = (jax.lax.broadcasted_iota(jnp.int32, (_C, _H), 0) * _H
         + jax.lax.broadcasted_iota(jnp.int32, (_C, _H), 1))
  wid = jax.lax.broadcasted_iota(jnp.int32, (1, _W), 1)

  def body(k, _):
    rm = rowmax_ref[...]
    m = jnp.max(rm)
    r = jnp.min(jnp.where(rm == m, rid, _BIG))
    row = nms_ref[pl.ds(r, 1), :]
    w = jnp.min(jnp.where(row == m, wid, _BIG))
    c = r // _H
    h = r - c * _H
    wmask = wid == w
    tag = jnp.sum(jnp.where(wmask, embd_ref[pl.ds(h, 1), :], 0.0))
    ox = jnp.sum(jnp.where(wmask, offx_ref[pl.ds(h, 1), :], 0.0))
    oy = jnp.sum(jnp.where(wmask, offy_ref[pl.ds(h, 1), :], 0.0))
    xf = w.astype(jnp.float32) + ox
    yf = h.astype(jnp.float32) + oy
    cf = c.astype(jnp.float32)
    attr_col_ref[pl.ds(k, 1), 0:1] = jnp.reshape(m, (1, 1))
    attr_col_ref[pl.ds(k, 1), 1:2] = jnp.reshape(cf, (1, 1))
    attr_col_ref[pl.ds(k, 1), 2:3] = jnp.reshape(tag, (1, 1))
    attr_col_ref[pl.ds(k, 1), 3:4] = jnp.reshape(xf, (1, 1))
    attr_col_ref[pl.ds(k, 1), 4:5] = jnp.reshape(yf, (1, 1))
    nms_ref[pl.ds(r, 1), :] = jnp.where(wmask, -1.0, row)
    newmax = jnp.max(jnp.where(wmask, -1.0, row))
    hid = jax.lax.broadcasted_iota(jnp.int32, (1, _H), 1)
    rmrow = rowmax_ref[pl.ds(c, 1), :]
    rowmax_ref[pl.ds(c, 1), :] = jnp.where(hid == h, newmax, rmrow)
    return 0

  jax.lax.fori_loop(0, _K, body, 0)


def _decoder_kernel(tlh_ref, brh_ref, tle_ref, bre_ref, tlox_ref, tloy_ref,
                    brox_ref, broy_ref, out_ref,
                    nms_ref, rowmax_ref, tl_col_ref, br_col_ref,
                    s_ref, f_ref, rvec_ref, cvec_ref, svec_ref):
  _phase1(tlh_ref, tle_ref, tlox_ref, tloy_ref, tl_col_ref,
          nms_ref, rowmax_ref)
  _phase1(brh_ref, bre_ref, brox_ref, broy_ref, br_col_ref,
          nms_ref, rowmax_ref)

  ts = tl_col_ref[:, 0:1]
  tcls = tl_col_ref[:, 1:2]
  ttag = tl_col_ref[:, 2:3]
  tx = tl_col_ref[:, 3:4]
  ty = tl_col_ref[:, 4:5]
  br_t = jnp.transpose(br_col_ref[...])
  bs = br_t[0:1, :]
  bcls = br_t[1:2, :]
  btag = br_t[2:3, :]
  bx = br_t[3:4, :]
  by = br_t[4:5, :]

  ii = jax.lax.broadcasted_iota(jnp.int32, (_H, _W), 0)
  jj = jax.lax.broadcasted_iota(jnp.int32, (_H, _W), 1)
  pad = (ii >= _K) | (jj >= _K)
  score = (ts + bs) * 0.5
  invalid = ((jnp.abs(ttag - btag) > _AE_THRESH)
             | (tcls != bcls) | (tx > bx) | (ty > by))
  f_ref[...] = jnp.where(invalid & ~pad, 1.0, 0.0)
  s_ref[...] = jnp.where(pad, -2.0, jnp.where(invalid, -1.0, score))

  lin = ii * _W + jj
  wid = jax.lax.broadcasted_iota(jnp.int32, (1, _W), 1)

  def vcond(carry):
    p, m = carry
    return (p < _NUM_DETS) & (m > -1.0)

  def vbody(carry):
    p, m = carry
    sv = s_ref[...]
    l = jnp.min(jnp.where(sv == m, lin, _BIG))
    r = l // _W
    c = l - r * _W
    rvec_ref[pl.ds(p, 1), 0:1] = jnp.reshape(r, (1, 1))
    cvec_ref[pl.ds(p, 1), 0:1] = jnp.reshape(c, (1, 1))
    svec_ref[pl.ds(p, 1), 0:1] = jnp.reshape(m, (1, 1))
    srow = s_ref[pl.ds(r, 1), :]
    s_ref[pl.ds(r, 1), :] = jnp.where(wid == c, -2.0, srow)
    return p + 1, jnp.max(s_ref[...])

  p0, _ = jax.lax.while_loop(vcond, vbody, (jnp.int32(0),
                                            jnp.max(s_ref[...])))

  def fcond(carry):
    p, l = carry
    return (p < _NUM_DETS) & (l < _H * _W)

  def fbody(carry):
    p, l = carry
    r = l // _W
    c = l - r * _W
    frow = f_ref[pl.ds(r, 1), :]
    take = jnp.sum(jnp.where(wid == c, frow, 0.0)) > 0.5

    @pl.when(take)
    def _():
      rvec_ref[pl.ds(p, 1), 0:1] = jnp.reshape(r, (1, 1))
      cvec_ref[pl.ds(p, 1), 0:1] = jnp.reshape(c, (1, 1))
      svec_ref[pl.ds(p, 1), 0:1] = jnp.full((1, 1), -1.0, jnp.float32)

    return p + take.astype(jnp.int32), l + 1

  jax.lax.while_loop(fcond, fbody, (p0, jnp.int32(0)))

  npad = out_ref.shape[0]
  rv = rvec_ref[...]
  cv = cvec_ref[...]
  sv = svec_ref[...]
  lane = jax.lax.broadcasted_iota(jnp.int32, (npad, _W), 1)
  oh_r = (rv == lane).astype(jnp.float32)
  oh_c = (cv == lane).astype(jnp.float32)

  lane8 = jax.lax.broadcasted_iota(jnp.int32, (_H, 8), 1)
  zero8 = jnp.zeros((_H, 8), jnp.float32)
  tl_mat = jnp.where(lane8 == 0, jnp.broadcast_to(tx, (_H, 8)), zero8)
  tl_mat = jnp.where(lane8 == 1, jnp.broadcast_to(ty, (_H, 8)), tl_mat)
  tl_mat = jnp.where(lane8 == 5, jnp.broadcast_to(ts, (_H, 8)), tl_mat)
  tl_mat = jnp.where(lane8 == 7, jnp.broadcast_to(tcls, (_H, 8)), tl_mat)
  bxc = br_col_ref[:, 3:4]
  byc = br_col_ref[:, 4:5]
  bsc = br_col_ref[:, 0:1]
  br_mat = jnp.where(lane8 == 2, jnp.broadcast_to(bxc, (_H, 8)), zero8)
  br_mat = jnp.where(lane8 == 3, jnp.broadcast_to(byc, (_H, 8)), br_mat)
  br_mat = jnp.where(lane8 == 6, jnp.broadcast_to(bsc, (_H, 8)), br_mat)
  row8 = jax.lax.broadcasted_iota(jnp.int32, (_H, 8), 0)
  tl_mat = jnp.where(row8 < _K, tl_mat, 0.0)
  br_mat = jnp.where(row8 < _K, br_mat, 0.0)

  det = (jnp.dot(oh_r, tl_mat, preferred_element_type=jnp.float32)
         + jnp.dot(oh_c, br_mat, preferred_element_type=jnp.float32))
  lane8b = jax.lax.broadcasted_iota(jnp.int32, (npad, 8), 1)
  det = jnp.where(lane8b == 4, jnp.broadcast_to(sv, (npad, 8)), det)
  out_ref[...] = det


@jax.jit
def kernel(tl_heat, br_heat, tl_embd, br_embd, tl_offs, br_offs):
  tlh = tl_heat[0]
  brh = br_heat[0]
  tle = tl_embd[0, 0]
  bre = br_embd[0, 0]
  tlox, tloy = tl_offs[0, 0], tl_offs[0, 1]
  brox, broy = br_offs[0, 0], br_offs[0, 1]
  npad = 1024
  out = pl.pallas_call(
      _decoder_kernel,
      out_shape=jax.ShapeDtypeStruct((npad, 8), jnp.float32),
      scratch_shapes=[
          pltpu.VMEM((_C * _H, _W), jnp.float32),
          pltpu.VMEM((_C, _H), jnp.float32),
          pltpu.VMEM((_H, 8), jnp.float32),
          pltpu.VMEM((_H, 8), jnp.float32),
          pltpu.VMEM((_H, _W), jnp.float32),
          pltpu.VMEM((_H, _W), jnp.float32),
          pltpu.VMEM((npad, 1), jnp.int32),
          pltpu.VMEM((npad, 1), jnp.int32),
          pltpu.VMEM((npad, 1), jnp.float32),
      ],
  )(tlh, brh, tle, bre, tlox, tloy, brox, broy)
  return out[:_NUM_DETS]
